# baseline (speedup 1.0000x reference)
import functools

import jax
import jax.numpy as jnp
from jax import lax
from jax.experimental import pallas as pl
from jax.experimental.pallas import tpu as pltpu
from jax.experimental.pallas import tpu_sc as plsc

_A = 0.12
_B = 2.6
_C = 1.6
_S = 64.0

_BATCH = 1024
_VOCAB = 100000
_NW = 32
_RPW = _BATCH // _NW
_W = 1408
_NCH = 71
_COLS_SC = _NCH * _W
_TAIL = _VOCAB - _COLS_SC


def _sqrt_sc(z):
    i = lax.bitcast_convert_type(z, jnp.int32)
    i = jnp.int32(0x5F3759DF) - lax.shift_right_logical(i, 1)
    y = lax.bitcast_convert_type(i, jnp.float32)
    for _ in range(3):
        y = y * (jnp.float32(1.5) - jnp.float32(0.5) * z * y * y)
    return z * y


def _acos(x, sqrt_fn):
    ax = jnp.abs(x)
    p = jnp.float32(-0.0012624911)
    p = p * ax + jnp.float32(0.0066700901)
    p = p * ax + jnp.float32(-0.0170881256)
    p = p * ax + jnp.float32(0.0308918810)
    p = p * ax + jnp.float32(-0.0501743046)
    p = p * ax + jnp.float32(0.0889789874)
    p = p * ax + jnp.float32(-0.2145988016)
    p = p * ax + jnp.float32(1.5707963050)
    r = sqrt_fn(jnp.maximum(jnp.float32(1.0) - ax, jnp.float32(0.0))) * p
    return jnp.where(x >= 0, r, jnp.float32(3.14159265358979) - r)


def _transform(x, sqrt_fn):
    t = _acos(x, sqrt_fn) + jnp.float32(_B)
    return jnp.float32(-_A) * (t * t) + jnp.float32(_C)


def _sc_body(cos_hbm, lab_hbm, out_hbm, lab_v, buf):
    wid = lax.axis_index("s") * 2 + lax.axis_index("c")
    r0 = pl.multiple_of(wid * _RPW, _RPW)
    pltpu.sync_copy(lab_hbm.at[pl.ds(r0, _RPW)], lab_v)

    def _group(gg, carry):
        rg = pl.multiple_of(r0 + gg * 16, 16)
        lv = lab_v[pl.ds(pl.multiple_of(gg * 16, 16), 16)]

        def _chunk(c, carry2):
            c0 = pl.multiple_of(c * _W, 128)
            pltpu.sync_copy(cos_hbm.at[pl.ds(rg, 16), pl.ds(c0, _W)], buf)

            for q in range(16):
                lab = lv[q]

                @pl.when(jnp.logical_and(lab >= c0, lab < c0 + _W))
                def _():
                    j = lab - c0
                    j_al = pl.multiple_of(
                        lax.shift_left(lax.shift_right_logical(j, 4), 4), 16)
                    vec = buf[q, pl.ds(j_al, 16)]
                    lane = lax.iota(jnp.int32, 16) + j_al
                    tgt = _transform(vec, _sqrt_sc)
                    buf[q, pl.ds(j_al, 16)] = jnp.where(lane == j, tgt, vec)

            def _scale(k, carry3):
                for q in range(16):
                    buf[q, pl.ds(k * 16, 16)] = (
                        buf[q, pl.ds(k * 16, 16)] * jnp.float32(_S))
                return carry3

            lax.fori_loop(0, _W // 16, _scale, 0)
            pltpu.sync_copy(buf, out_hbm.at[pl.ds(rg, 16), pl.ds(c0, _W)])
            return carry2

        lax.fori_loop(0, _NCH, _chunk, 0)
        return carry

    lax.fori_loop(0, _RPW // 16, _group, 0)


def _tc_tail_body(prev_ref, strip_ref, lab_ref, out_ref):
    strip = strip_ref[...]
    lab = lab_ref[...]
    cols = _COLS_SC + jax.lax.broadcasted_iota(jnp.int32, strip.shape, 1)
    hit = cols == lab
    orig = jnp.sum(jnp.where(hit, strip, 0.0), axis=1, keepdims=True)
    tgt = _transform(orig, jnp.sqrt)
    out_ref[:, : _TAIL] = jnp.where(hit, tgt, strip) * _S


def kernel(cosine, label):
    mesh = plsc.VectorSubcoreMesh(core_axis_name="c", subcore_axis_name="s")
    sc = functools.partial(
        pl.kernel,
        mesh=mesh,
        out_type=jax.ShapeDtypeStruct((_BATCH, _VOCAB), jnp.float32),
        scratch_types=[
            pltpu.VMEM((_RPW,), jnp.int32),
            pltpu.VMEM((16, _W), jnp.float32),
        ],
    )(_sc_body)
    tmp = sc(cosine, label)

    strip = cosine[:, _COLS_SC:]
    lab2 = label.reshape(_BATCH, 1)
    return pl.pallas_call(
        _tc_tail_body,
        grid=(1,),
        in_specs=[
            pl.BlockSpec(memory_space=pl.ANY),
            pl.BlockSpec((_BATCH, _TAIL), lambda i: (0, 0)),
            pl.BlockSpec((_BATCH, 1), lambda i: (0, 0)),
        ],
        out_specs=pl.BlockSpec((_BATCH, 128), lambda i: (0, _COLS_SC // 128)),
        out_shape=jax.ShapeDtypeStruct((_BATCH, _VOCAB), jnp.float32),
        input_output_aliases={0: 0},
    )(tmp, strip, lab2)


# speedup vs baseline: 1.3469x; 1.3469x over previous
import functools

import jax
import jax.numpy as jnp
from jax import lax
from jax.experimental import pallas as pl
from jax.experimental.pallas import tpu as pltpu
from jax.experimental.pallas import tpu_sc as plsc

_A = 0.12
_B = 2.6
_C = 1.6
_S = 64.0

_BATCH = 1024
_VOCAB = 100000
_NW = 32
_RPW = _BATCH // _NW
_W = 1408
_NCH = 71
_COLS_SC = _NCH * _W
_TAIL = _VOCAB - _COLS_SC


def _sqrt_sc(z):
    i = lax.bitcast_convert_type(z, jnp.int32)
    i = jnp.int32(0x5F3759DF) - lax.shift_right_logical(i, 1)
    y = lax.bitcast_convert_type(i, jnp.float32)
    for _ in range(3):
        y = y * (jnp.float32(1.5) - jnp.float32(0.5) * z * y * y)
    return z * y


def _acos(x, sqrt_fn):
    ax = jnp.abs(x)
    p = jnp.float32(-0.0012624911)
    p = p * ax + jnp.float32(0.0066700901)
    p = p * ax + jnp.float32(-0.0170881256)
    p = p * ax + jnp.float32(0.0308918810)
    p = p * ax + jnp.float32(-0.0501743046)
    p = p * ax + jnp.float32(0.0889789874)
    p = p * ax + jnp.float32(-0.2145988016)
    p = p * ax + jnp.float32(1.5707963050)
    r = sqrt_fn(jnp.maximum(jnp.float32(1.0) - ax, jnp.float32(0.0))) * p
    return jnp.where(x >= 0, r, jnp.float32(3.14159265358979) - r)


def _transform(x, sqrt_fn):
    t = _acos(x, sqrt_fn) + jnp.float32(_B)
    return jnp.float32(-_A) * (t * t) + jnp.float32(_C)


_TOT = 2 * _NCH


def _sc_body(cos_hbm, lab_hbm, out_hbm, lab_v, x0, x1, y0, y1, si0, si1, so0, so1):
    wid = lax.axis_index("s") * 2 + lax.axis_index("c")
    r0 = pl.multiple_of(wid * _RPW, _RPW)
    pltpu.sync_copy(lab_hbm.at[pl.ds(r0, _RPW)], lab_v)
    xs, ys, sis, sos = (x0, x1), (y0, y1), (si0, si1), (so0, so1)

    def _slc(tt):
        gg = lax.div(tt, _NCH)
        c = lax.rem(tt, _NCH)
        rg = pl.multiple_of(r0 + gg * 16, 16)
        c0 = pl.multiple_of(c * _W, 128)
        return gg, rg, c0

    def _in_copy(tt, b):
        _, rg, c0 = _slc(tt)
        return pltpu.make_async_copy(
            cos_hbm.at[pl.ds(rg, 16), pl.ds(c0, _W)], xs[b], sis[b])

    def _out_copy(tt, b):
        _, rg, c0 = _slc(tt)
        return pltpu.make_async_copy(
            ys[b], out_hbm.at[pl.ds(rg, 16), pl.ds(c0, _W)], sos[b])

    _in_copy(jnp.int32(0), 0).start()
    _in_copy(jnp.int32(1), 1).start()

    def _iter(i, carry):
        for b in range(2):
            tt = 2 * i + b
            gg, rg, c0 = _slc(tt)
            _in_copy(tt, b).wait()

            @pl.when(tt >= 2)
            def _():
                _out_copy(tt - 2, b).wait()

            lv = lab_v[pl.ds(pl.multiple_of(gg * 16, 16), 16)]
            for q in range(16):
                lab = lv[q]

                @pl.when(jnp.logical_and(lab >= c0, lab < c0 + _W))
                def _():
                    j = lab - c0
                    j_al = pl.multiple_of(
                        lax.shift_left(lax.shift_right_logical(j, 4), 4), 16)
                    vec = xs[b][q, pl.ds(j_al, 16)]
                    lane = lax.iota(jnp.int32, 16) + j_al
                    tgt = _transform(vec, _sqrt_sc)
                    xs[b][q, pl.ds(j_al, 16)] = jnp.where(lane == j, tgt, vec)

            def _scale(k, carry3):
                for q in range(16):
                    ys[b][q, pl.ds(k * 16, 16)] = (
                        xs[b][q, pl.ds(k * 16, 16)] * jnp.float32(_S))
                return carry3

            lax.fori_loop(0, _W // 16, _scale, 0)
            _out_copy(tt, b).start()

            @pl.when(tt + 2 < _TOT)
            def _():
                _in_copy(tt + 2, b).start()
        return carry

    lax.fori_loop(0, _TOT // 2, _iter, 0)
    for b in range(2):
        _out_copy(jnp.int32(_TOT - 2 + b), b).wait()


def _tc_tail_body(prev_ref, strip_ref, lab_ref, out_ref):
    strip = strip_ref[...]
    lab = lab_ref[...]
    cols = _COLS_SC + jax.lax.broadcasted_iota(jnp.int32, strip.shape, 1)
    hit = cols == lab
    orig = jnp.sum(jnp.where(hit, strip, 0.0), axis=1, keepdims=True)
    tgt = _transform(orig, jnp.sqrt)
    out_ref[:, : _TAIL] = jnp.where(hit, tgt, strip) * _S


def kernel(cosine, label):
    mesh = plsc.VectorSubcoreMesh(core_axis_name="c", subcore_axis_name="s")
    sc = functools.partial(
        pl.kernel,
        mesh=mesh,
        out_type=jax.ShapeDtypeStruct((_BATCH, _VOCAB), jnp.float32),
        scratch_types=[
            pltpu.VMEM((_RPW,), jnp.int32),
            pltpu.VMEM((16, _W), jnp.float32),
            pltpu.VMEM((16, _W), jnp.float32),
            pltpu.VMEM((16, _W), jnp.float32),
            pltpu.VMEM((16, _W), jnp.float32),
            pltpu.SemaphoreType.DMA,
            pltpu.SemaphoreType.DMA,
            pltpu.SemaphoreType.DMA,
            pltpu.SemaphoreType.DMA,
        ],
    )(_sc_body)
    tmp = sc(cosine, label)

    strip = cosine[:, _COLS_SC:]
    lab2 = label.reshape(_BATCH, 1)
    return pl.pallas_call(
        _tc_tail_body,
        grid=(1,),
        in_specs=[
            pl.BlockSpec(memory_space=pl.ANY),
            pl.BlockSpec((_BATCH, _TAIL), lambda i: (0, 0)),
            pl.BlockSpec((_BATCH, 1), lambda i: (0, 0)),
        ],
        out_specs=pl.BlockSpec((_BATCH, 128), lambda i: (0, _COLS_SC // 128)),
        out_shape=jax.ShapeDtypeStruct((_BATCH, _VOCAB), jnp.float32),
        input_output_aliases={0: 0},
    )(tmp, strip, lab2)


# speedup vs baseline: 3.4573x; 2.5668x over previous
import functools

import jax
import jax.numpy as jnp
from jax import lax
from jax.experimental import pallas as pl
from jax.experimental.pallas import tpu as pltpu
from jax.experimental.pallas import tpu_sc as plsc

_A = 0.12
_B = 2.6
_C = 1.6
_S = 64.0

_BATCH = 1024
_VOCAB = 100000
_NW = 32
_UNITS = _VOCAB // 8
_BASE_U = _UNITS // _NW
_EXTRA = _UNITS % _NW
_NCHUNK = (_BASE_U * 8) // 16


def _sqrt_sc(z):
    i = lax.bitcast_convert_type(z, jnp.int32)
    i = jnp.int32(0x5F3759DF) - lax.shift_right_logical(i, 1)
    y = lax.bitcast_convert_type(i, jnp.float32)
    for _ in range(3):
        y = y * (jnp.float32(1.5) - jnp.float32(0.5) * z * y * y)
    return z * y


def _acos(x, sqrt_fn):
    ax = jnp.abs(x)
    p = jnp.float32(-0.0012624911)
    p = p * ax + jnp.float32(0.0066700901)
    p = p * ax + jnp.float32(-0.0170881256)
    p = p * ax + jnp.float32(0.0308918810)
    p = p * ax + jnp.float32(-0.0501743046)
    p = p * ax + jnp.float32(0.0889789874)
    p = p * ax + jnp.float32(-0.2145988016)
    p = p * ax + jnp.float32(1.5707963050)
    r = sqrt_fn(jnp.maximum(jnp.float32(1.0) - ax, jnp.float32(0.0))) * p
    return jnp.where(x >= 0, r, jnp.float32(3.14159265358979) - r)


def _transform(x, sqrt_fn):
    t = _acos(x, sqrt_fn) + jnp.float32(_B)
    return jnp.float32(-_A) * (t * t) + jnp.float32(_C)


def _sc_body(cos_hbm, lab_hbm, out_hbm,
             lab_all, x0, x1, y0, y1, rw, cb, si0, si1, so0, so1):
    wid = lax.axis_index("s") * 2 + lax.axis_index("c")
    nu = _BASE_U + jnp.where(wid < _EXTRA, 1, 0)
    start_row = pl.multiple_of((wid * _BASE_U + jnp.minimum(wid, _EXTRA)) * 8, 8)
    pltpu.sync_copy(lab_hbm, lab_all)
    xs, ys, sis, sos = (x0, x1), (y0, y1), (si0, si1), (so0, so1)

    def _row(tt):
        return pl.multiple_of(start_row + tt * 16, 8)

    def _in_copy(tt, b):
        return pltpu.make_async_copy(
            cos_hbm.at[pl.ds(_row(tt), 16)], xs[b], sis[b])

    def _out_copy(tt, b):
        return pltpu.make_async_copy(
            ys[b], out_hbm.at[pl.ds(_row(tt), 16)], sos[b])

    def _scale_chunk(b):
        def _scale(k, carry):
            for q in range(16):
                ys[b][q, pl.ds(k * 16, 16)] = (
                    xs[b][q, pl.ds(k * 16, 16)] * jnp.float32(_S))
            return carry

        lax.fori_loop(0, _BATCH // 16, _scale, 0)

    _in_copy(jnp.int32(0), 0).start()
    _in_copy(jnp.int32(1), 1).start()

    def _iter(i, carry):
        for b in range(2):
            tt = 2 * i + b
            _in_copy(tt, b).wait()

            @pl.when(tt >= 2)
            def _():
                _out_copy(tt - 2, b).wait()

            _scale_chunk(b)
            _out_copy(tt, b).start()

            @pl.when(tt + 2 < _NCHUNK)
            def _():
                _in_copy(tt + 2, b).start()
        return carry

    lax.fori_loop(0, _NCHUNK // 2, _iter, 0)

    _in_copy(jnp.int32(_NCHUNK - 1), 0).wait()
    _out_copy(jnp.int32(_NCHUNK - 3), 0).wait()
    _scale_chunk(0)
    _out_copy(jnp.int32(_NCHUNK - 1), 0).start()
    _out_copy(jnp.int32(_NCHUNK - 2), 1).wait()
    _out_copy(jnp.int32(_NCHUNK - 1), 0).wait()

    @pl.when(wid < _EXTRA)
    def _():
        r8 = pl.multiple_of(start_row + _NCHUNK * 16, 8)
        pltpu.sync_copy(cos_hbm.at[pl.ds(r8, 8)], rw)

        def _scale8(k, carry):
            for q in range(8):
                rw[q, pl.ds(k * 16, 16)] = (
                    rw[q, pl.ds(k * 16, 16)] * jnp.float32(_S))
            return carry

        lax.fori_loop(0, _BATCH // 16, _scale8, 0)
        pltpu.sync_copy(rw, out_hbm.at[pl.ds(r8, 8)])

    vlo = start_row
    vhi = start_row + nu * 8
    iota16 = lax.iota(jnp.int32, 16)

    def _patch_group(g, carry):
        lv = lab_all[pl.ds(g * 16, 16)]
        for q in range(16):
            v = lv[q]
            @pl.when(jnp.logical_and(v >= vlo, v < vhi))
            def _():
                pltpu.sync_copy(out_hbm.at[v, pl.ds(g * 16, 16)], cb)
                v16 = cb[...]
                orig = v16 * jnp.float32(1.0 / _S)
                t16 = _transform(orig, _sqrt_sc) * jnp.float32(_S)
                cb[...] = jnp.where(iota16 == q, t16, v16)
                pltpu.sync_copy(cb, out_hbm.at[v, pl.ds(g * 16, 16)])
        return carry

    lax.fori_loop(0, _BATCH // 16, _patch_group, 0)


def kernel(cosine, label):
    cos_t = cosine.T
    mesh = plsc.VectorSubcoreMesh(core_axis_name="c", subcore_axis_name="s")
    sc = functools.partial(
        pl.kernel,
        mesh=mesh,
        out_type=jax.ShapeDtypeStruct((_VOCAB, _BATCH), jnp.float32),
        scratch_types=[
            pltpu.VMEM((_BATCH,), jnp.int32),
            pltpu.VMEM((16, _BATCH), jnp.float32),
            pltpu.VMEM((16, _BATCH), jnp.float32),
            pltpu.VMEM((16, _BATCH), jnp.float32),
            pltpu.VMEM((16, _BATCH), jnp.float32),
            pltpu.VMEM((8, _BATCH), jnp.float32),
            pltpu.VMEM((16,), jnp.float32),
            pltpu.SemaphoreType.DMA,
            pltpu.SemaphoreType.DMA,
            pltpu.SemaphoreType.DMA,
            pltpu.SemaphoreType.DMA,
        ],
    )(_sc_body)
    return sc(cos_t, label).T


# speedup vs baseline: 4.0866x; 1.1820x over previous
import functools

import jax
import jax.numpy as jnp
from jax import lax
from jax.experimental import pallas as pl
from jax.experimental.pallas import tpu as pltpu
from jax.experimental.pallas import tpu_sc as plsc

_A = 0.12
_B = 2.6
_C = 1.6
_S = 64.0

_BATCH = 1024
_VOCAB = 100000
_NW = 32
_UNITS = _VOCAB // 8
_BASE_U = _UNITS // _NW
_EXTRA = _UNITS % _NW
_NCHUNK = (_BASE_U * 8) // 16


def _sqrt_sc(z):
    i = lax.bitcast_convert_type(z, jnp.int32)
    i = jnp.int32(0x5F3759DF) - lax.shift_right_logical(i, 1)
    y = lax.bitcast_convert_type(i, jnp.float32)
    for _ in range(3):
        y = y * (jnp.float32(1.5) - jnp.float32(0.5) * z * y * y)
    return z * y


def _acos(x, sqrt_fn):
    ax = jnp.abs(x)
    p = jnp.float32(-0.0012624911)
    p = p * ax + jnp.float32(0.0066700901)
    p = p * ax + jnp.float32(-0.0170881256)
    p = p * ax + jnp.float32(0.0308918810)
    p = p * ax + jnp.float32(-0.0501743046)
    p = p * ax + jnp.float32(0.0889789874)
    p = p * ax + jnp.float32(-0.2145988016)
    p = p * ax + jnp.float32(1.5707963050)
    r = sqrt_fn(jnp.maximum(jnp.float32(1.0) - ax, jnp.float32(0.0))) * p
    return jnp.where(x >= 0, r, jnp.float32(3.14159265358979) - r)


def _transform(x, sqrt_fn):
    t = _acos(x, sqrt_fn) + jnp.float32(_B)
    return jnp.float32(-_A) * (t * t) + jnp.float32(_C)


def _sc_body(cos_hbm, lab_hbm, out_hbm,
             lab_all, x0, x1, y0, y1, rw, cb, si0, si1, so0, so1):
    wid = lax.axis_index("s") * 2 + lax.axis_index("c")
    nu = _BASE_U + jnp.where(wid < _EXTRA, 1, 0)
    start_row = pl.multiple_of((wid * _BASE_U + jnp.minimum(wid, _EXTRA)) * 8, 8)
    pltpu.sync_copy(lab_hbm, lab_all)
    xs, ys, sis, sos = (x0, x1), (y0, y1), (si0, si1), (so0, so1)

    def _row(tt):
        return pl.multiple_of(start_row + tt * 16, 8)

    def _in_copy(tt, b):
        return pltpu.make_async_copy(
            cos_hbm.at[pl.ds(_row(tt), 16)], xs[b], sis[b])

    def _out_copy(tt, b):
        return pltpu.make_async_copy(
            ys[b], out_hbm.at[pl.ds(_row(tt), 16)], sos[b])

    def _scale_chunk(b):
        @plsc.parallel_loop(0, _BATCH // 16, unroll=4)
        def _scale(k):
            for q in range(16):
                ys[b][q, pl.ds(k * 16, 16)] = (
                    xs[b][q, pl.ds(k * 16, 16)] * jnp.float32(_S))

    _in_copy(jnp.int32(0), 0).start()
    _in_copy(jnp.int32(1), 1).start()

    def _iter(i, carry):
        for b in range(2):
            tt = 2 * i + b
            _in_copy(tt, b).wait()

            @pl.when(tt >= 2)
            def _():
                _out_copy(tt - 2, b).wait()

            _scale_chunk(b)
            _out_copy(tt, b).start()

            @pl.when(tt + 2 < _NCHUNK)
            def _():
                _in_copy(tt + 2, b).start()
        return carry

    lax.fori_loop(0, _NCHUNK // 2, _iter, 0)

    _in_copy(jnp.int32(_NCHUNK - 1), 0).wait()
    _out_copy(jnp.int32(_NCHUNK - 3), 0).wait()
    _scale_chunk(0)
    _out_copy(jnp.int32(_NCHUNK - 1), 0).start()
    _out_copy(jnp.int32(_NCHUNK - 2), 1).wait()
    _out_copy(jnp.int32(_NCHUNK - 1), 0).wait()

    @pl.when(wid < _EXTRA)
    def _():
        r8 = pl.multiple_of(start_row + _NCHUNK * 16, 8)
        pltpu.sync_copy(cos_hbm.at[pl.ds(r8, 8)], rw)

        def _scale8(k, carry):
            for q in range(8):
                rw[q, pl.ds(k * 16, 16)] = (
                    rw[q, pl.ds(k * 16, 16)] * jnp.float32(_S))
            return carry

        lax.fori_loop(0, _BATCH // 16, _scale8, 0)
        pltpu.sync_copy(rw, out_hbm.at[pl.ds(r8, 8)])

    vlo = start_row
    vhi = start_row + nu * 8
    iota16 = lax.iota(jnp.int32, 16)

    def _patch_group(g, carry):
        lv = lab_all[pl.ds(g * 16, 16)]
        for q in range(16):
            v = lv[q]
            @pl.when(jnp.logical_and(v >= vlo, v < vhi))
            def _():
                pltpu.sync_copy(out_hbm.at[v, pl.ds(g * 16, 16)], cb)
                v16 = cb[...]
                orig = v16 * jnp.float32(1.0 / _S)
                t16 = _transform(orig, _sqrt_sc) * jnp.float32(_S)
                cb[...] = jnp.where(iota16 == q, t16, v16)
                pltpu.sync_copy(cb, out_hbm.at[v, pl.ds(g * 16, 16)])
        return carry

    lax.fori_loop(0, _BATCH // 16, _patch_group, 0)


def kernel(cosine, label):
    cos_t = cosine.T
    mesh = plsc.VectorSubcoreMesh(core_axis_name="c", subcore_axis_name="s")
    sc = functools.partial(
        pl.kernel,
        mesh=mesh,
        out_type=jax.ShapeDtypeStruct((_VOCAB, _BATCH), jnp.float32),
        scratch_types=[
            pltpu.VMEM((_BATCH,), jnp.int32),
            pltpu.VMEM((16, _BATCH), jnp.float32),
            pltpu.VMEM((16, _BATCH), jnp.float32),
            pltpu.VMEM((16, _BATCH), jnp.float32),
            pltpu.VMEM((16, _BATCH), jnp.float32),
            pltpu.VMEM((8, _BATCH), jnp.float32),
            pltpu.VMEM((16,), jnp.float32),
            pltpu.SemaphoreType.DMA,
            pltpu.SemaphoreType.DMA,
            pltpu.SemaphoreType.DMA,
            pltpu.SemaphoreType.DMA,
        ],
    )(_sc_body)
    return sc(cos_t, label).T
